# baseline (speedup 1.0000x reference)
import jax
import jax.numpy as jnp
from jax import lax
from jax.experimental import pallas as pl
from jax.experimental.pallas import tpu as pltpu
from jax.experimental.pallas import tpu_sc as plsc

_ALPHA = 0.5
_EPSILON = 10.0

_NUM_CORES = 2
_NUM_SUBCORES = 16
_NW = _NUM_CORES * _NUM_SUBCORES

_N_CHUNKS = 8


def _scan_kernel(ps_ref, pt_ref, a_ref):
    ps = ps_ref[...]
    t_dim, n_dim = ps.shape
    x = jnp.concatenate(
        [jnp.ones((t_dim, 1), jnp.float32), 1.0 - ps[:, 1:]], axis=1
    )
    s = 1
    while s < n_dim:
        x = x * jnp.concatenate(
            [jnp.ones((t_dim, s), jnp.float32), x[:, :-s]], axis=1
        )
        s *= 2
    ps_next = jnp.concatenate(
        [ps[:, 1:], jnp.ones((t_dim, 1), jnp.float32)], axis=1
    )
    pt = ps_next * x + _EPSILON / t_dim
    t_col = lax.broadcasted_iota(jnp.int32, (t_dim, n_dim), 0).astype(jnp.float32)
    pt_ref[...] = pt
    a_ref[...] = jnp.reshape(pt * (1.0 - t_col / t_dim), (-1,))


def _sc_perm_body(a_hbm, idx_hbm, out_hbm, idx_v, w_v, sem):
    chunk = idx_v.shape[0]
    wid = lax.axis_index("s") * _NUM_CORES + lax.axis_index("c")
    base = wid * chunk
    pltpu.sync_copy(idx_hbm.at[pl.ds(base, chunk)], idx_v)
    pltpu.async_copy(a_hbm.at[idx_v], w_v, sem).wait()
    pltpu.sync_copy(w_v, out_hbm.at[pl.ds(base, chunk)])


def _gather_kernel(lcp_ref, pt_ref, y_ref, expg_ref, cls_ref, acc):
    i = pl.program_id(0)
    lcp = lcp_ref[...]
    yt = jnp.transpose(y_ref[...])
    c_iota = lax.broadcasted_iota(jnp.int32, lcp.shape, 2)
    g = jnp.sum(
        jnp.where(c_iota == yt[:, :, None], lcp, 0.0), axis=2
    )

    @pl.when(i == 0)
    def _():
        acc[0] = 0.0

    acc[0] += jnp.sum(g * pt_ref[...])
    expg_ref[...] = jnp.exp(g)

    @pl.when(i == pl.num_programs(0) - 1)
    def _():
        cls_ref[...] = jnp.broadcast_to(acc[0], (1, 1))


def _earl_kernel(expg_ref, w_ref, out_ref):
    e = jnp.reshape(expg_ref[...], (-1,))
    out_ref[...] = jnp.broadcast_to(jnp.sum(e * w_ref[...]), (1, 1))


def kernel(log_class_probabilities, probability_stopping, y_true):
    T, N, C = log_class_probabilities.shape
    M = T * N
    chunk = M // _NW
    nc = N // _N_CHUNKS

    pt, a_flat = pl.pallas_call(
        _scan_kernel,
        out_shape=[
            jax.ShapeDtypeStruct((T, N), jnp.float32),
            jax.ShapeDtypeStruct((M,), jnp.float32),
        ],
    )(probability_stopping)

    j = lax.iota(jnp.int32, M)
    perm_idx = (j % T) * N + j // T

    sc_perm = pl.kernel(
        _sc_perm_body,
        mesh=plsc.VectorSubcoreMesh(core_axis_name="c", subcore_axis_name="s"),
        out_type=jax.ShapeDtypeStruct((M,), jnp.float32),
        scratch_types=[
            pltpu.VMEM((chunk,), jnp.int32),
            pltpu.VMEM((chunk,), jnp.float32),
            pltpu.SemaphoreType.DMA,
        ],
    )
    w_flat = sc_perm(a_flat, perm_idx)

    expg, cls_g = pl.pallas_call(
        _gather_kernel,
        grid=(_N_CHUNKS,),
        in_specs=[
            pl.BlockSpec((T, nc, C), lambda i: (0, i, 0)),
            pl.BlockSpec((T, nc), lambda i: (0, i)),
            pl.BlockSpec((nc, T), lambda i: (i, 0)),
        ],
        out_specs=[
            pl.BlockSpec((T, nc), lambda i: (0, i)),
            pl.BlockSpec((1, 1), lambda i: (0, 0)),
        ],
        out_shape=[
            jax.ShapeDtypeStruct((T, N), jnp.float32),
            jax.ShapeDtypeStruct((1, 1), jnp.float32),
        ],
        scratch_shapes=[pltpu.SMEM((1,), jnp.float32)],
    )(log_class_probabilities, pt, y_true)

    earl = pl.pallas_call(
        _earl_kernel,
        out_shape=jax.ShapeDtypeStruct((1, 1), jnp.float32),
    )(expg, w_flat)

    cls = cls_g[0, 0]
    return (_ALPHA * (-cls) - (1.0 - _ALPHA) * earl[0, 0]) / T


# speedup vs baseline: 1.5664x; 1.5664x over previous
import jax
import jax.numpy as jnp
from jax import lax
from jax.experimental import pallas as pl
from jax.experimental.pallas import tpu as pltpu
from jax.experimental.pallas import tpu_sc as plsc

_ALPHA = 0.5
_EPSILON = 10.0

_NUM_CORES = 2
_NUM_SUBCORES = 16
_NW = _NUM_CORES * _NUM_SUBCORES


def _scan_kernel(ps_ref, y_ref, pt_ref, a_ref, idx_ref):
    ps = ps_ref[...]
    t_dim, n_dim = ps.shape
    c_dim = idx_ref.shape[0] // (t_dim * n_dim)
    x = jnp.concatenate(
        [jnp.ones((t_dim, 1), jnp.float32), 1.0 - ps[:, 1:]], axis=1
    )
    s = 1
    while s < n_dim:
        x = x * jnp.concatenate(
            [jnp.ones((t_dim, s), jnp.float32), x[:, :-s]], axis=1
        )
        s *= 2
    ps_next = jnp.concatenate(
        [ps[:, 1:], jnp.ones((t_dim, 1), jnp.float32)], axis=1
    )
    pt = ps_next * x + _EPSILON / t_dim
    t_iota = lax.broadcasted_iota(jnp.int32, (t_dim, n_dim), 0)
    n_iota = lax.broadcasted_iota(jnp.int32, (t_dim, n_dim), 1)
    yt = jnp.transpose(y_ref[...])
    pt_ref[...] = jnp.reshape(pt, (-1,))
    a_ref[...] = jnp.reshape(
        pt * (1.0 - t_iota.astype(jnp.float32) / t_dim), (-1,)
    )
    idx_ref[...] = jnp.reshape((t_iota * n_dim + n_iota) * c_dim + yt, (-1,))


def _sc_body(lcp_hbm, idx_hbm, a_hbm, pidx_hbm, g_out, w_out,
             idx_v, g_v, pidx_v, w_v, sem_g, sem_w):
    chunk = idx_v.shape[0]
    wid = lax.axis_index("s") * _NUM_CORES + lax.axis_index("c")
    base = wid * chunk
    pltpu.sync_copy(idx_hbm.at[pl.ds(base, chunk)], idx_v)
    cp_g = pltpu.async_copy(lcp_hbm.at[idx_v], g_v, sem_g)
    pltpu.sync_copy(pidx_hbm.at[pl.ds(base, chunk)], pidx_v)
    cp_w = pltpu.async_copy(a_hbm.at[pidx_v], w_v, sem_w)
    cp_g.wait()
    pltpu.sync_copy(g_v, g_out.at[pl.ds(base, chunk)])
    cp_w.wait()
    pltpu.sync_copy(w_v, w_out.at[pl.ds(base, chunk)])


def _make_reduce_kernel(t_dim):
    def _reduce_kernel(g_ref, w_ref, pt_ref, out_ref):
        m = g_ref.shape[0]
        g = jnp.reshape(g_ref[...], (m // 128, 128))
        w = jnp.reshape(w_ref[...], (m // 128, 128))
        pt = jnp.reshape(pt_ref[...], (m // 128, 128))
        cls_g = jnp.sum(g * pt)
        earl = jnp.sum(jnp.exp(g) * w)
        loss = (_ALPHA * (-cls_g) - (1.0 - _ALPHA) * earl) / t_dim
        out_ref[...] = jnp.broadcast_to(loss, (1, 1))

    return _reduce_kernel


def kernel(log_class_probabilities, probability_stopping, y_true):
    T, N, C = log_class_probabilities.shape
    M = T * N
    chunk = M // _NW

    lcp_flat = jnp.reshape(log_class_probabilities, (-1,))

    pt_flat, a_flat, idx_flat = pl.pallas_call(
        _scan_kernel,
        out_shape=[
            jax.ShapeDtypeStruct((M,), jnp.float32),
            jax.ShapeDtypeStruct((M,), jnp.float32),
            jax.ShapeDtypeStruct((M,), jnp.int32),
        ],
    )(probability_stopping, y_true)

    j = lax.iota(jnp.int32, M)
    perm_idx = (j % T) * N + j // T

    sc_gather = pl.kernel(
        _sc_body,
        mesh=plsc.VectorSubcoreMesh(core_axis_name="c", subcore_axis_name="s"),
        out_type=[
            jax.ShapeDtypeStruct((M,), jnp.float32),
            jax.ShapeDtypeStruct((M,), jnp.float32),
        ],
        scratch_types=[
            pltpu.VMEM((chunk,), jnp.int32),
            pltpu.VMEM((chunk,), jnp.float32),
            pltpu.VMEM((chunk,), jnp.int32),
            pltpu.VMEM((chunk,), jnp.float32),
            pltpu.SemaphoreType.DMA,
            pltpu.SemaphoreType.DMA,
        ],
    )
    g_flat, w_flat = sc_gather(lcp_flat, idx_flat, a_flat, perm_idx)

    loss = pl.pallas_call(
        _make_reduce_kernel(T),
        out_shape=jax.ShapeDtypeStruct((1, 1), jnp.float32),
    )(g_flat, w_flat, pt_flat)

    return loss[0, 0]


# speedup vs baseline: 4.5561x; 2.9087x over previous
import jax
import jax.numpy as jnp
from jax import lax
from jax.experimental import pallas as pl
from jax.experimental.pallas import tpu as pltpu
from jax.experimental.pallas import tpu_sc as plsc

_ALPHA = 0.5
_EPSILON = 10.0

_NUM_CORES = 2
_NUM_SUBCORES = 16
_NW = _NUM_CORES * _NUM_SUBCORES

_N_CHUNKS = 8


def _scan_kernel(ps_ref, pt_ref, a_ref):
    ps = ps_ref[...]
    t_dim, n_dim = ps.shape
    x = jnp.concatenate(
        [jnp.ones((t_dim, 1), jnp.float32), 1.0 - ps[:, 1:]], axis=1
    )
    s = 1
    while s < n_dim:
        x = x * jnp.concatenate(
            [jnp.ones((t_dim, s), jnp.float32), x[:, :-s]], axis=1
        )
        s *= 2
    ps_next = jnp.concatenate(
        [ps[:, 1:], jnp.ones((t_dim, 1), jnp.float32)], axis=1
    )
    pt = ps_next * x + _EPSILON / t_dim
    t_col = lax.broadcasted_iota(jnp.int32, (t_dim, n_dim), 0).astype(jnp.float32)
    pt_ref[...] = pt
    a_ref[...] = jnp.reshape(pt * (1.0 - t_col / t_dim), (-1,))


def _sc_perm_body(a_hbm, idx_hbm, out_hbm, idx_v, w_v, sem):
    chunk = idx_v.shape[0]
    wid = lax.axis_index("s") * _NUM_CORES + lax.axis_index("c")
    base = wid * chunk
    pltpu.sync_copy(idx_hbm.at[pl.ds(base, chunk)], idx_v)
    pltpu.async_copy(a_hbm.at[idx_v], w_v, sem).wait()
    pltpu.sync_copy(w_v, out_hbm.at[pl.ds(base, chunk)])


def _gather_kernel(lcp_ref, pt_ref, y_ref, expg_ref, cls_ref, acc):
    i = pl.program_id(0)
    lcp = lcp_ref[...]
    yt = jnp.transpose(y_ref[...])
    c_iota = lax.broadcasted_iota(jnp.int32, lcp.shape, 1)
    g = jnp.sum(
        jnp.where(c_iota == yt[:, None, :], lcp, 0.0), axis=1
    )

    @pl.when(i == 0)
    def _():
        acc[0] = 0.0

    acc[0] += jnp.sum(g * pt_ref[...])
    expg_ref[...] = jnp.exp(g)

    @pl.when(i == pl.num_programs(0) - 1)
    def _():
        cls_ref[...] = jnp.broadcast_to(acc[0], (1, 1))


def _make_earl_kernel(t_dim):
    def _earl_kernel(expg_ref, w_ref, cls_ref, out_ref):
        e = jnp.reshape(expg_ref[...], (-1,))
        earl = jnp.sum(e * w_ref[...])
        cls_g = cls_ref[0, 0]
        loss = (_ALPHA * (-cls_g) - (1.0 - _ALPHA) * earl) / t_dim
        out_ref[...] = jnp.broadcast_to(loss, (1, 1))

    return _earl_kernel


def kernel(log_class_probabilities, probability_stopping, y_true):
    T, N, C = log_class_probabilities.shape
    M = T * N
    chunk = M // _NW
    nc = N // _N_CHUNKS

    lcp_t = jnp.transpose(log_class_probabilities, (0, 2, 1))

    pt, a_flat = pl.pallas_call(
        _scan_kernel,
        out_shape=[
            jax.ShapeDtypeStruct((T, N), jnp.float32),
            jax.ShapeDtypeStruct((M,), jnp.float32),
        ],
    )(probability_stopping)

    j = lax.iota(jnp.int32, M)
    perm_idx = (j % T) * N + j // T

    sc_perm = pl.kernel(
        _sc_perm_body,
        mesh=plsc.VectorSubcoreMesh(core_axis_name="c", subcore_axis_name="s"),
        out_type=jax.ShapeDtypeStruct((M,), jnp.float32),
        scratch_types=[
            pltpu.VMEM((chunk,), jnp.int32),
            pltpu.VMEM((chunk,), jnp.float32),
            pltpu.SemaphoreType.DMA,
        ],
    )
    w_flat = sc_perm(a_flat, perm_idx)

    expg, cls_g = pl.pallas_call(
        _gather_kernel,
        grid=(_N_CHUNKS,),
        in_specs=[
            pl.BlockSpec((T, C, nc), lambda i: (0, 0, i)),
            pl.BlockSpec((T, nc), lambda i: (0, i)),
            pl.BlockSpec((nc, T), lambda i: (i, 0)),
        ],
        out_specs=[
            pl.BlockSpec((T, nc), lambda i: (0, i)),
            pl.BlockSpec((1, 1), lambda i: (0, 0)),
        ],
        out_shape=[
            jax.ShapeDtypeStruct((T, N), jnp.float32),
            jax.ShapeDtypeStruct((1, 1), jnp.float32),
        ],
        scratch_shapes=[pltpu.SMEM((1,), jnp.float32)],
    )(lcp_t, pt, y_true)

    loss = pl.pallas_call(
        _make_earl_kernel(T),
        out_shape=jax.ShapeDtypeStruct((1, 1), jnp.float32),
    )(expg, w_flat, cls_g)

    return loss[0, 0]
